# baseline (speedup 1.0000x reference)
import functools

import numpy as np

import jax
import jax.numpy as jnp
from jax import lax
from jax.experimental import pallas as pl
from jax.experimental.pallas import tpu as pltpu
from jax.experimental.pallas import tpu_sc as plsc

B, L, S = 1024, 50, 4
D_EMB, D_MODEL = 128, 512
NC, NS, LANES = 2, 16, 16
NW = NC * NS
BPW = B // NW
ROWS = BPW * S
VCH = D_EMB // LANES
NP = 5
LP = L // NP


def _sc_gather_sum(lp0, idx_hbm, table_hbm, feat_hbm, idx_v,
                   rows0, rows1, feat0, feat1, gsem0, gsem1, osem0, osem1):
    w = lax.axis_index("s") * NC + lax.axis_index("c")
    rows, feat, gsem, osem = (rows0, rows1), (feat0, feat1), (gsem0, gsem1), (osem0, osem1)
    pltpu.sync_copy(idx_hbm.at[:, pl.ds(w * ROWS, ROWS)], idx_v)

    def gather(l, bi):
        return pltpu.make_async_copy(
            table_hbm.at[idx_v.at[lp0 + l]], rows[bi], gsem[bi])

    def outcopy(l, bi):
        return pltpu.make_async_copy(
            feat[bi], feat_hbm.at[l, pl.ds(w * BPW, BPW)], osem[bi])

    def chunk(l, bi):
        gather(l, bi).wait()

        @pl.when(l >= 2)
        def _():
            outcopy(l - 2, bi).wait()

        def tok_body(t, tc):
            r = 4 * t
            rv = rows[bi]
            for k in range(VCH // 2):
                sl0 = pl.ds(2 * LANES * k, LANES)
                sl1 = pl.ds(2 * LANES * k + LANES, LANES)
                a = rv[r, sl0] + rv[r + 1, sl0] + rv[r + 2, sl0] + rv[r + 3, sl0]
                b = rv[r, sl1] + rv[r + 1, sl1] + rv[r + 2, sl1] + rv[r + 3, sl1]
                feat[bi][t, pl.ds(LANES * k, LANES)] = plsc.bitcast(
                    plsc.pack(a, b, format=plsc.PackFormat.INTERLEAVED),
                    jnp.float32,
                )
            return tc

        lax.fori_loop(0, BPW, tok_body, 0)
        outcopy(l, bi).start()

    gather(0, 0).start()

    def outer_body(i, carry):
        for bi in range(2):
            l = 2 * i + bi

            @pl.when(l + 1 < LP)
            def _():
                gather(l + 1, 1 - bi).start()

            chunk(l, bi)
        return carry

    lax.fori_loop(0, LP // 2, outer_body, 0)
    if LP % 2:
        chunk(LP - 1, (LP - 1) % 2)
    outcopy(LP - 2, (LP - 2) % 2).wait()
    outcopy(LP - 1, (LP - 1) % 2).wait()


def _make_sc_call(lp0):
    return functools.partial(
        pl.kernel,
        out_type=jax.ShapeDtypeStruct((LP, B, D_EMB // 2), jnp.float32),
        mesh=plsc.VectorSubcoreMesh(core_axis_name="c", subcore_axis_name="s"),
        compiler_params=pltpu.CompilerParams(
            use_tc_tiling_on_sc=True, needs_layout_passes=False),
        scratch_types=[
            pltpu.VMEM((L, ROWS), jnp.int32),
            pltpu.VMEM((ROWS, D_EMB), jnp.float32),
            pltpu.VMEM((ROWS, D_EMB), jnp.float32),
            pltpu.VMEM((BPW, D_EMB // 2), jnp.float32),
            pltpu.VMEM((BPW, D_EMB // 2), jnp.float32),
            pltpu.SemaphoreType.DMA,
            pltpu.SemaphoreType.DMA,
            pltpu.SemaphoreType.DMA,
            pltpu.SemaphoreType.DMA,
        ],
    )(functools.partial(_sc_gather_sum, lp0))


_sc_calls = [_make_sc_call(p * LP) for p in range(NP)]


_BB = 32
_MM_BLK = LP * _BB


_BLK = np.empty(2 * LANES, dtype=np.int32)
_BLK[0::2] = np.arange(LANES)
_BLK[1::2] = LANES + np.arange(LANES)
_PERM = np.concatenate([2 * LANES * k + _BLK for k in range(VCH // 2)])


def _mm_compute(f_ref, w_ref, b_ref, o_ref):
    f = f_ref[...].reshape(_MM_BLK, D_EMB).astype(jnp.float32)
    m = jnp.dot(f, w_ref[...], preferred_element_type=jnp.float32)
    o_ref[...] = m.reshape(LP, _BB, D_MODEL) + b_ref[...]


def _mm_body0(f_ref, w_ref, b_ref, o_ref):
    _mm_compute(f_ref, w_ref, b_ref, o_ref)


def _mm_body1(f_ref, w_ref, b_ref, prev_ref, o_ref):
    del prev_ref
    _mm_compute(f_ref, w_ref, b_ref, o_ref)


def _make_out_index_map(p):
    return lambda i: (p, i, 0)


def _tc_matmul(feats, W, b3):
    f_spec = pl.BlockSpec((LP, _BB, D_EMB), lambda i: (0, i, 0))
    w_spec = pl.BlockSpec((D_EMB, D_MODEL), lambda i: (0, 0))
    b_spec = pl.BlockSpec((1, 1, D_MODEL), lambda i: (0, 0, 0))
    out_shape = jax.ShapeDtypeStruct((L, B, D_MODEL), jnp.float32)

    out = pl.pallas_call(
        _mm_body0,
        grid=(B // _BB,),
        in_specs=[f_spec, w_spec, b_spec],
        out_specs=pl.BlockSpec((LP, _BB, D_MODEL), _make_out_index_map(0)),
        out_shape=out_shape,
    )(feats[0], W, b3)
    for p in range(1, NP):
        out = pl.pallas_call(
            _mm_body1,
            grid=(B // _BB,),
            in_specs=[f_spec, w_spec, b_spec,
                      pl.BlockSpec(memory_space=pl.ANY)],
            out_specs=pl.BlockSpec((LP, _BB, D_MODEL), _make_out_index_map(p)),
            out_shape=out_shape,
            input_output_aliases={3: 0},
        )(feats[p], W, b3, out)
    return out


def kernel(indices, table, W, b):
    idx = jnp.transpose(indices.astype(jnp.int32), (1, 0, 2)).reshape(L, B * S)
    feats = [
        jax.lax.bitcast_convert_type(call(idx, table), jnp.bfloat16)
        .reshape(LP, B, D_EMB)
        for call in _sc_calls
    ]
    W2 = W[jnp.asarray(_PERM)]
    out = _tc_matmul(feats, W2, b.reshape(1, 1, D_MODEL))
    return (jnp.transpose(out, (1, 0, 2)), None)


# speedup vs baseline: 1.7363x; 1.7363x over previous
import functools

import jax
import jax.numpy as jnp
from jax import lax
from jax.experimental import pallas as pl
from jax.experimental.pallas import tpu as pltpu
from jax.experimental.pallas import tpu_sc as plsc

B, L, S = 1024, 50, 4
D_EMB, D_MODEL = 128, 512
NC, NS, LANES = 2, 16, 16
NW = NC * NS
BPW = B // NW
ROWS = BPW * S
VCH = D_EMB // LANES
NP = 5
LP = L // NP


def _sc_gather_sum(lp0, idx_hbm, table_hbm, feat_hbm, idx_v,
                   rows0, rows1, feat0, feat1, gsem0, gsem1, osem0, osem1):
    w = lax.axis_index("s") * NC + lax.axis_index("c")
    rows, feat, gsem, osem = (rows0, rows1), (feat0, feat1), (gsem0, gsem1), (osem0, osem1)
    pltpu.sync_copy(idx_hbm.at[:, pl.ds(w * ROWS, ROWS)], idx_v)

    def gather(l, bi):
        return pltpu.make_async_copy(
            table_hbm.at[idx_v.at[lp0 + l]], rows[bi], gsem[bi])

    def outcopy(l, bi):
        return pltpu.make_async_copy(
            feat[bi], feat_hbm.at[l, pl.ds(w * (BPW // 2), BPW // 2)], osem[bi])

    def chunk(l, bi):
        gather(l, bi).wait()

        @pl.when(l >= 2)
        def _():
            outcopy(l - 2, bi).wait()

        def tok_body(t, tc):
            r = 8 * t
            rv = rows[bi]
            for h in range(VCH):
                sl = pl.ds(LANES * h, LANES)
                a = rv[r, sl] + rv[r + 1, sl] + rv[r + 2, sl] + rv[r + 3, sl]
                b = rv[r + 4, sl] + rv[r + 5, sl] + rv[r + 6, sl] + rv[r + 7, sl]
                feat[bi][t, sl] = plsc.bitcast(
                    plsc.pack(a, b, format=plsc.PackFormat.INTERLEAVED),
                    jnp.float32,
                )
            return tc

        lax.fori_loop(0, BPW // 2, tok_body, 0)
        outcopy(l, bi).start()

    gather(0, 0).start()

    def outer_body(i, carry):
        for bi in range(2):
            l = 2 * i + bi

            @pl.when(l + 1 < LP)
            def _():
                gather(l + 1, 1 - bi).start()

            chunk(l, bi)
        return carry

    lax.fori_loop(0, LP // 2, outer_body, 0)
    if LP % 2:
        chunk(LP - 1, (LP - 1) % 2)
    outcopy(LP - 2, (LP - 2) % 2).wait()
    outcopy(LP - 1, (LP - 1) % 2).wait()


def _make_sc_call(lp0):
    return functools.partial(
        pl.kernel,
        out_type=jax.ShapeDtypeStruct((LP, B // 2, D_EMB), jnp.float32),
        mesh=plsc.VectorSubcoreMesh(core_axis_name="c", subcore_axis_name="s"),
        compiler_params=pltpu.CompilerParams(
            use_tc_tiling_on_sc=True, needs_layout_passes=False),
        scratch_types=[
            pltpu.VMEM((L, ROWS), jnp.int32),
            pltpu.VMEM((ROWS, D_EMB), jnp.float32),
            pltpu.VMEM((ROWS, D_EMB), jnp.float32),
            pltpu.VMEM((BPW // 2, D_EMB), jnp.float32),
            pltpu.VMEM((BPW // 2, D_EMB), jnp.float32),
            pltpu.SemaphoreType.DMA,
            pltpu.SemaphoreType.DMA,
            pltpu.SemaphoreType.DMA,
            pltpu.SemaphoreType.DMA,
        ],
    )(functools.partial(_sc_gather_sum, lp0))


_sc_calls = [_make_sc_call(p * LP) for p in range(NP)]


_BB = 32
_MM_BLK = LP * _BB


def _mm_compute(f_ref, w_ref, b_ref, o_ref):
    f = pltpu.bitcast(f_ref[...].reshape(_MM_BLK // 2, D_EMB), jnp.bfloat16)
    m = jnp.dot(f, w_ref[...], preferred_element_type=jnp.float32)
    o_ref[...] = m.reshape(LP, _BB, D_MODEL) + b_ref[...]


def _mm_body0(f_ref, w_ref, b_ref, o_ref):
    _mm_compute(f_ref, w_ref, b_ref, o_ref)


def _mm_body1(f_ref, w_ref, b_ref, prev_ref, o_ref):
    del prev_ref
    _mm_compute(f_ref, w_ref, b_ref, o_ref)


def _make_out_index_map(p):
    return lambda i: (p, i, 0)


def _tc_matmul(feats, W, b3):
    f_spec = pl.BlockSpec((LP, _BB // 2, D_EMB), lambda i: (0, i, 0))
    w_spec = pl.BlockSpec((D_EMB, D_MODEL), lambda i: (0, 0))
    b_spec = pl.BlockSpec((1, 1, D_MODEL), lambda i: (0, 0, 0))
    out_shape = jax.ShapeDtypeStruct((L, B, D_MODEL), jnp.float32)

    out = pl.pallas_call(
        _mm_body0,
        grid=(B // _BB,),
        in_specs=[f_spec, w_spec, b_spec],
        out_specs=pl.BlockSpec((LP, _BB, D_MODEL), _make_out_index_map(0)),
        out_shape=out_shape,
    )(feats[0], W, b3)
    for p in range(1, NP):
        out = pl.pallas_call(
            _mm_body1,
            grid=(B // _BB,),
            in_specs=[f_spec, w_spec, b_spec,
                      pl.BlockSpec(memory_space=pl.ANY)],
            out_specs=pl.BlockSpec((LP, _BB, D_MODEL), _make_out_index_map(p)),
            out_shape=out_shape,
            input_output_aliases={3: 0},
        )(feats[p], W, b3, out)
    return out


def kernel(indices, table, W, b):
    idx = jnp.transpose(indices.astype(jnp.int32), (1, 0, 2)).reshape(L, B * S)
    feats = [call(idx, table) for call in _sc_calls]
    out = _tc_matmul(feats, W, b.reshape(1, 1, D_MODEL))
    return (jnp.transpose(out, (1, 0, 2)), None)


# speedup vs baseline: 1.9354x; 1.1147x over previous
import functools

import jax
import jax.numpy as jnp
from jax import lax
from jax.experimental import pallas as pl
from jax.experimental.pallas import tpu as pltpu
from jax.experimental.pallas import tpu_sc as plsc

B, L, S = 1024, 50, 4
D_EMB, D_MODEL = 128, 512
NC, NS, LANES = 2, 16, 16
NW = NC * NS
BPW = B // NW
ROWS = BPW * S
VCH = D_EMB // LANES
NP = 5
LP = L // NP


def _sc_gather_sum(lp0, idx_hbm, table_hbm, feat_hbm, idx_v,
                   rows0, rows1, feat0, feat1, gsem0, gsem1, osem0, osem1):
    w = lax.axis_index("s") * NC + lax.axis_index("c")
    rows, feat, gsem, osem = (rows0, rows1), (feat0, feat1), (gsem0, gsem1), (osem0, osem1)
    pltpu.sync_copy(idx_hbm.at[:, pl.ds(w * ROWS, ROWS)], idx_v)

    def gather(l, bi):
        return pltpu.make_async_copy(
            table_hbm.at[idx_v.at[lp0 + l]], rows[bi], gsem[bi])

    def outcopy(l, bi):
        return pltpu.make_async_copy(
            feat[bi], feat_hbm.at[l, pl.ds(w * (BPW // 2), BPW // 2)], osem[bi])

    def chunk(l, bi):
        gather(l, bi).wait()

        @pl.when(l >= 2)
        def _():
            outcopy(l - 2, bi).wait()

        def tok_body(t, tc):
            r = 8 * t
            rv = rows[bi]
            for h in range(VCH):
                sl = pl.ds(LANES * h, LANES)
                a = rv[r, sl] + rv[r + 1, sl] + rv[r + 2, sl] + rv[r + 3, sl]
                b = rv[r + 4, sl] + rv[r + 5, sl] + rv[r + 6, sl] + rv[r + 7, sl]
                feat[bi][t, sl] = plsc.bitcast(
                    plsc.pack(a, b, format=plsc.PackFormat.INTERLEAVED),
                    jnp.float32,
                )
            return tc

        lax.fori_loop(0, BPW // 2, tok_body, 0)
        outcopy(l, bi).start()

    gather(0, 0).start()

    def outer_body(i, carry):
        for bi in range(2):
            l = 2 * i + bi

            @pl.when(l + 1 < LP)
            def _():
                gather(l + 1, 1 - bi).start()

            chunk(l, bi)
        return carry

    lax.fori_loop(0, LP // 2, outer_body, 0)
    if LP % 2:
        chunk(LP - 1, (LP - 1) % 2)
    outcopy(LP - 2, (LP - 2) % 2).wait()
    outcopy(LP - 1, (LP - 1) % 2).wait()


def _make_sc_call(lp0):
    return functools.partial(
        pl.kernel,
        out_type=jax.ShapeDtypeStruct((LP, B // 2, D_EMB), jnp.float32),
        mesh=plsc.VectorSubcoreMesh(core_axis_name="c", subcore_axis_name="s"),
        compiler_params=pltpu.CompilerParams(
            use_tc_tiling_on_sc=True, needs_layout_passes=False),
        scratch_types=[
            pltpu.VMEM((L, ROWS), jnp.int32),
            pltpu.VMEM((ROWS, D_EMB), jnp.float32),
            pltpu.VMEM((ROWS, D_EMB), jnp.float32),
            pltpu.VMEM((BPW // 2, D_EMB), jnp.float32),
            pltpu.VMEM((BPW // 2, D_EMB), jnp.float32),
            pltpu.SemaphoreType.DMA,
            pltpu.SemaphoreType.DMA,
            pltpu.SemaphoreType.DMA,
            pltpu.SemaphoreType.DMA,
        ],
    )(functools.partial(_sc_gather_sum, lp0))


_sc_calls = [_make_sc_call(p * LP) for p in range(NP)]


_BB = 64
_MM_BLK = LP * _BB


def _mm_compute(f_ref, w_ref, b_ref, o_ref):
    f = pltpu.bitcast(f_ref[...].reshape(_MM_BLK // 2, D_EMB), jnp.bfloat16)
    m = jnp.dot(f, w_ref[...], preferred_element_type=jnp.float32)
    o_ref[...] = m.reshape(LP, _BB, D_MODEL) + b_ref[...]


def _mm_body0(f_ref, w_ref, b_ref, o_ref):
    _mm_compute(f_ref, w_ref, b_ref, o_ref)


def _mm_body1(f_ref, w_ref, b_ref, prev_ref, o_ref):
    del prev_ref
    _mm_compute(f_ref, w_ref, b_ref, o_ref)


def _make_out_index_map(p):
    return lambda i: (p, i, 0)


def _tc_matmul(feats, W, b3):
    f_spec = pl.BlockSpec((LP, _BB // 2, D_EMB), lambda i: (0, i, 0))
    w_spec = pl.BlockSpec((D_EMB, D_MODEL), lambda i: (0, 0))
    b_spec = pl.BlockSpec((1, 1, D_MODEL), lambda i: (0, 0, 0))
    out_shape = jax.ShapeDtypeStruct((L, B, D_MODEL), jnp.float32)

    out = pl.pallas_call(
        _mm_body0,
        grid=(B // _BB,),
        in_specs=[f_spec, w_spec, b_spec],
        out_specs=pl.BlockSpec((LP, _BB, D_MODEL), _make_out_index_map(0)),
        out_shape=out_shape,
    )(feats[0], W, b3)
    for p in range(1, NP):
        out = pl.pallas_call(
            _mm_body1,
            grid=(B // _BB,),
            in_specs=[f_spec, w_spec, b_spec,
                      pl.BlockSpec(memory_space=pl.ANY)],
            out_specs=pl.BlockSpec((LP, _BB, D_MODEL), _make_out_index_map(p)),
            out_shape=out_shape,
            input_output_aliases={3: 0},
        )(feats[p], W, b3, out)
    return out


def kernel(indices, table, W, b):
    idx = jnp.transpose(indices.astype(jnp.int32), (1, 0, 2)).reshape(L, B * S)
    feats = [call(idx, table) for call in _sc_calls]
    out = _tc_matmul(feats, W, b.reshape(1, 1, D_MODEL))
    return (jnp.transpose(out, (1, 0, 2)), None)


# speedup vs baseline: 1.9574x; 1.0113x over previous
import functools

import jax
import jax.numpy as jnp
from jax import lax
from jax.experimental import pallas as pl
from jax.experimental.pallas import tpu as pltpu
from jax.experimental.pallas import tpu_sc as plsc

B, L, S = 1024, 50, 4
D_EMB, D_MODEL = 128, 512
NC, NS, LANES = 2, 16, 16
NW = NC * NS
BPW = B // NW
ROWS = BPW * S
VCH = D_EMB // LANES
NP = 5
LP = L // NP


def _sc_gather_sum(lp0, idx_hbm, table_hbm, feat_hbm, idx_v,
                   rows0, rows1, feat0, feat1, gsem0, gsem1, osem0, osem1):
    w = lax.axis_index("s") * NC + lax.axis_index("c")
    rows, feat, gsem, osem = (rows0, rows1), (feat0, feat1), (gsem0, gsem1), (osem0, osem1)
    pltpu.sync_copy(idx_hbm.at[:, pl.ds(w * ROWS, ROWS)], idx_v)

    def gather(l, bi):
        return pltpu.make_async_copy(
            table_hbm.at[idx_v.at[lp0 + l]], rows[bi], gsem[bi])

    def outcopy(l, bi):
        return pltpu.make_async_copy(
            feat[bi], feat_hbm.at[l, pl.ds(w * (BPW // 2), BPW // 2)], osem[bi])

    def chunk(l, bi):
        gather(l, bi).wait()

        @pl.when(l >= 2)
        def _():
            outcopy(l - 2, bi).wait()

        def tok_body(t, tc):
            r = 8 * t
            rv = rows[bi]
            for h in range(VCH):
                sl = pl.ds(LANES * h, LANES)
                a = rv[r, sl] + rv[r + 1, sl] + rv[r + 2, sl] + rv[r + 3, sl]
                b = rv[r + 4, sl] + rv[r + 5, sl] + rv[r + 6, sl] + rv[r + 7, sl]
                feat[bi][t, sl] = plsc.bitcast(
                    plsc.pack(a, b, format=plsc.PackFormat.INTERLEAVED),
                    jnp.float32,
                )
            return tc

        lax.fori_loop(0, BPW // 2, tok_body, 0)
        outcopy(l, bi).start()

    gather(0, 0).start()

    def outer_body(i, carry):
        for bi in range(2):
            l = 2 * i + bi

            @pl.when(l + 1 < LP)
            def _():
                gather(l + 1, 1 - bi).start()

            chunk(l, bi)
        return carry

    lax.fori_loop(0, LP // 2, outer_body, 0)
    if LP % 2:
        chunk(LP - 1, (LP - 1) % 2)
    outcopy(LP - 2, (LP - 2) % 2).wait()
    outcopy(LP - 1, (LP - 1) % 2).wait()


def _make_sc_call(lp0):
    return functools.partial(
        pl.kernel,
        out_type=jax.ShapeDtypeStruct((LP, B // 2, D_EMB), jnp.float32),
        mesh=plsc.VectorSubcoreMesh(core_axis_name="c", subcore_axis_name="s"),
        compiler_params=pltpu.CompilerParams(
            use_tc_tiling_on_sc=True, needs_layout_passes=False),
        scratch_types=[
            pltpu.VMEM((L, ROWS), jnp.int32),
            pltpu.VMEM((ROWS, D_EMB), jnp.float32),
            pltpu.VMEM((ROWS, D_EMB), jnp.float32),
            pltpu.VMEM((BPW // 2, D_EMB), jnp.float32),
            pltpu.VMEM((BPW // 2, D_EMB), jnp.float32),
            pltpu.SemaphoreType.DMA,
            pltpu.SemaphoreType.DMA,
            pltpu.SemaphoreType.DMA,
            pltpu.SemaphoreType.DMA,
        ],
    )(functools.partial(_sc_gather_sum, lp0))


_sc_calls = [_make_sc_call(p * LP) for p in range(NP)]


_BB = 128
_MM_BLK = LP * _BB


def _mm_compute(f_ref, w_ref, b_ref, o_ref):
    f = pltpu.bitcast(f_ref[...].reshape(_MM_BLK // 2, D_EMB), jnp.bfloat16)
    m = jnp.dot(f, w_ref[...], preferred_element_type=jnp.float32)
    o_ref[...] = m.reshape(LP, _BB, D_MODEL) + b_ref[...]


def _mm_body0(f_ref, w_ref, b_ref, o_ref):
    _mm_compute(f_ref, w_ref, b_ref, o_ref)


def _mm_body1(f_ref, w_ref, b_ref, prev_ref, o_ref):
    del prev_ref
    _mm_compute(f_ref, w_ref, b_ref, o_ref)


def _make_out_index_map(p):
    return lambda i: (p, i, 0)


def _tc_matmul(feats, W, b3):
    f_spec = pl.BlockSpec((LP, _BB // 2, D_EMB), lambda i: (0, i, 0))
    w_spec = pl.BlockSpec((D_EMB, D_MODEL), lambda i: (0, 0))
    b_spec = pl.BlockSpec((1, 1, D_MODEL), lambda i: (0, 0, 0))
    out_shape = jax.ShapeDtypeStruct((L, B, D_MODEL), jnp.float32)

    out = pl.pallas_call(
        _mm_body0,
        grid=(B // _BB,),
        in_specs=[f_spec, w_spec, b_spec],
        out_specs=pl.BlockSpec((LP, _BB, D_MODEL), _make_out_index_map(0)),
        out_shape=out_shape,
    )(feats[0], W, b3)
    for p in range(1, NP):
        out = pl.pallas_call(
            _mm_body1,
            grid=(B // _BB,),
            in_specs=[f_spec, w_spec, b_spec,
                      pl.BlockSpec(memory_space=pl.ANY)],
            out_specs=pl.BlockSpec((LP, _BB, D_MODEL), _make_out_index_map(p)),
            out_shape=out_shape,
            input_output_aliases={3: 0},
        )(feats[p], W, b3, out)
    return out


def kernel(indices, table, W, b):
    idx = jnp.transpose(indices.astype(jnp.int32), (1, 0, 2)).reshape(L, B * S)
    feats = [call(idx, table) for call in _sc_calls]
    out = _tc_matmul(feats, W, b.reshape(1, 1, D_MODEL))
    return (jnp.transpose(out, (1, 0, 2)), None)
